# baseline (speedup 1.0000x reference)
import functools

import jax
import jax.numpy as jnp
from jax import lax
from jax.experimental import pallas as pl
from jax.experimental.pallas import tpu as pltpu
from jax.experimental.pallas import tpu_sc as plsc

B, K, D, H, S, M = 32, 8192, 256, 256, 2, 64
TEMP = 0.1
KC = 1024
KB = K // KC
SCALE = 1.0 / 16.0

_HI = jax.lax.Precision.HIGHEST


def _bf(x):
    return x.astype(jnp.bfloat16)


def _dt(a, b):
    return jax.lax.dot_general(a, b, (((1,), (1,)), ((), ())), precision=_HI)


def _bdt(a, b):
    return jax.lax.dot_general(_bf(a), _bf(b), (((1,), (1,)), ((), ())),
                               preferred_element_type=jnp.float32)


def _bmulsum(a, b, axis):
    return jnp.sum(_bf(a).astype(jnp.float32) * _bf(b).astype(jnp.float32),
                   axis=axis)


def _l2k(x, axis=-1):
    n = jnp.sqrt(jnp.sum(x * x, axis=axis, keepdims=True))
    return x / jnp.maximum(n, 1e-12)


def _lnk(x, g, b):
    mu = jnp.mean(x, axis=-1, keepdims=True)
    xc = x - mu
    var = jnp.mean(xc * xc, axis=-1, keepdims=True)
    return xc / jnp.sqrt(var + 1e-5) * g + b


def _a_body(cand_ref, q_ref, inw_ref, inb_ref, cb_ref, s_ref, qp_scr):
    b = pl.program_id(0)
    kb = pl.program_id(1)

    @pl.when(kb == 0)
    def _init():
        qrow = q_ref[pl.ds(b, 1), :]
        qp_scr[...] = _dt(qrow, inw_ref[0:H, :]) + inb_ref[pl.ds(0, 1), 0:H]

    x = cand_ref[0]
    xb = _bf(x)
    cb_ref[0] = xb
    kp = jax.lax.dot_general(xb, _bf(inw_ref[H:2 * H, :]),
                             (((1,), (1,)), ((), ())),
                             preferred_element_type=jnp.float32) \
        + inb_ref[pl.ds(0, 1), H:2 * H]
    s = _bdt(qp_scr[...], kp) * SCALE
    s_ref[0, 0, :] = s[0]


def _run_a(cand, q, inw, inb):
    return pl.pallas_call(
        _a_body,
        grid=(B, KB),
        in_specs=[
            pl.BlockSpec((1, KC, D), lambda b, kb: (b, kb, 0)),
            pl.BlockSpec((B, D), lambda b, kb: (0, 0)),
            pl.BlockSpec((3 * H, H), lambda b, kb: (0, 0)),
            pl.BlockSpec((1, 3 * H), lambda b, kb: (0, 0)),
        ],
        out_specs=[
            pl.BlockSpec((1, KC, D), lambda b, kb: (b, kb, 0)),
            pl.BlockSpec((1, 1, KC), lambda b, kb: (b * KB + kb, 0, 0)),
        ],
        out_shape=[
            jax.ShapeDtypeStruct((B, K, D), jnp.bfloat16),
            jax.ShapeDtypeStruct((B * KB, 1, KC), jnp.float32),
        ],
        scratch_shapes=[pltpu.VMEM((1, H), jnp.float32)],
        compiler_params=pltpu.CompilerParams(
            dimension_semantics=("parallel", "arbitrary")),
    )(cand, q, inw, inb.reshape(1, 3 * H))


def _a2_body(srow_ref, sblk_ref, cb_ref, inw_ref, inb_ref, o_ref,
             ml_scr, acc_scr):
    kb = pl.program_id(1)

    @pl.when(kb == 0)
    def _init():
        srow = srow_ref[0, 0, :]
        m = jnp.max(srow)
        ml_scr[0] = m
        ml_scr[1] = jnp.sum(jnp.exp(srow - m))
        acc_scr[...] = jnp.zeros_like(acc_scr)

    att = jnp.exp(sblk_ref[0, 0, :] - ml_scr[0]) / ml_scr[1]
    vp = jax.lax.dot_general(cb_ref[0], _bf(inw_ref[2 * H:3 * H, :]),
                             (((1,), (1,)), ((), ())),
                             preferred_element_type=jnp.float32) \
        + inb_ref[pl.ds(0, 1), 2 * H:3 * H]
    acc_scr[...] += jax.lax.dot_general(
        _bf(att).reshape(1, KC), _bf(vp), (((1,), (0,)), ((), ())),
        preferred_element_type=jnp.float32)

    @pl.when(kb == KB - 1)
    def _fin():
        o_ref[0, 0, :] = acc_scr[0]


def _run_a2(s, cb, inw, inb):
    srow = s.reshape(B, 1, K)
    return pl.pallas_call(
        _a2_body,
        grid=(B, KB),
        in_specs=[
            pl.BlockSpec((1, 1, K), lambda b, kb: (b, 0, 0)),
            pl.BlockSpec((1, 1, KC), lambda b, kb: (b * KB + kb, 0, 0)),
            pl.BlockSpec((1, KC, D), lambda b, kb: (b, kb, 0)),
            pl.BlockSpec((3 * H, H), lambda b, kb: (0, 0)),
            pl.BlockSpec((1, 3 * H), lambda b, kb: (0, 0)),
        ],
        out_specs=pl.BlockSpec((1, 1, D), lambda b, kb: (b, 0, 0)),
        out_shape=jax.ShapeDtypeStruct((B, 1, D), jnp.float32),
        scratch_shapes=[
            pltpu.SMEM((2,), jnp.float32),
            pltpu.VMEM((1, H), jnp.float32),
        ],
        compiler_params=pltpu.CompilerParams(
            dimension_semantics=("parallel", "arbitrary")),
    )(srow, s, cb, inw, inb.reshape(1, 3 * H))


def _b_body(o_ref, q_ref, outw_ref, outb_ref, ng_ref, nb_ref,
            qpw_ref, se_ref, h_ref, hs_ref):
    o = o_ref[:, 0, :]
    a = _bdt(o, outw_ref[...]) + outb_ref[pl.ds(0, 1), :]
    x = a + q_ref[...]
    qe = _lnk(x, ng_ref[pl.ds(0, 1), :], nb_ref[pl.ds(0, 1), :])
    h = _l2k(_bdt(qe, qpw_ref[...]))
    hs = _l2k(h + se_ref[pl.ds(0, 1), :])
    h_ref[...] = h
    hs_ref[...] = hs


def _run_b(o, q, outw, outb, ng, nb, qpw, se):
    return pl.pallas_call(
        _b_body,
        out_shape=[jax.ShapeDtypeStruct((B, H), jnp.float32)] * 2,
    )(o, q, outw, outb.reshape(1, H), ng.reshape(1, H), nb.reshape(1, H),
      qpw, se)


def _c_body(cb_ref, hs_ref, cpw_ref, cheap_ref):
    p = jax.lax.dot_general(cb_ref[0], _bf(cpw_ref[...]),
                            (((1,), (1,)), ((), ())),
                            preferred_element_type=jnp.float32)
    n2 = jnp.sum(p * p, axis=1, keepdims=True)
    cpnb = _bf(p / jnp.maximum(jnp.sqrt(n2), 1e-12))
    hsb = _bf(hs_ref[0])
    dv = jax.lax.dot_general(hsb, cpnb, (((1,), (1,)), ((), ())),
                             preferred_element_type=jnp.float32)
    cheap_ref[0, 0, :] = dv[0] / TEMP


def _run_c(cb, hs, cpw):
    return pl.pallas_call(
        _c_body,
        grid=(B, KB),
        in_specs=[
            pl.BlockSpec((1, KC, D), lambda b, kb: (b, kb, 0)),
            pl.BlockSpec((1, 1, D), lambda b, kb: (b, 0, 0)),
            pl.BlockSpec((H, H), lambda b, kb: (0, 0)),
        ],
        out_specs=pl.BlockSpec((1, 1, KC), lambda b, kb: (b * KB + kb, 0, 0)),
        out_shape=jax.ShapeDtypeStruct((B * KB, 1, KC), jnp.float32),
        compiler_params=pltpu.CompilerParams(
            dimension_semantics=("parallel", "parallel")),
    )(cb, hs.reshape(B, 1, H), cpw)


def _t_body(cheap_ref, pp_ref, idx_ref, flat_ref):
    c = cheap_ref[...]
    ki = jax.lax.broadcasted_iota(jnp.int32, (B, K), 1)
    pp = pp_ref[:, 0:1]
    c = jnp.where(ki == pp, -1e9, c)
    cols = []
    for _ in range(M):
        v = jnp.max(c, axis=1, keepdims=True)
        eq = c == v
        im = jnp.min(jnp.where(eq, ki, K), axis=1, keepdims=True)
        cols.append(im)
        c = jnp.where(ki == im, -jnp.inf, c)
    idx = jnp.concatenate(cols, axis=1)
    idx_ref[...] = idx
    row = jax.lax.broadcasted_iota(jnp.int32, (B, M), 0)
    flat_ref[...] = idx + row * K


def _run_t(cheap, prev_pred):
    return pl.pallas_call(
        _t_body,
        out_shape=[
            jax.ShapeDtypeStruct((B, M), jnp.int32),
            jax.ShapeDtypeStruct((B, M), jnp.int32),
        ],
    )(cheap, prev_pred)


_G_CACHE = []


def _g_kernel(table, flat_idx):
    if not _G_CACHE:
        mesh = plsc.VectorSubcoreMesh(core_axis_name="c", subcore_axis_name="s")
        nc = mesh.num_cores

        @functools.partial(
            pl.kernel,
            mesh=mesh,
            out_type=jax.ShapeDtypeStruct((B, M, D), jnp.float32),
            scratch_types=[
                pltpu.VMEM((M,), jnp.int32),
                pltpu.VMEM((M, D), jnp.float32),
                pltpu.SemaphoreType.DMA,
            ],
        )
        def _g(table_hbm, idx_hbm, out_hbm, idx_v, rows_v, sem):
            wid = lax.axis_index("s") * nc + lax.axis_index("c")
            pltpu.sync_copy(idx_hbm.at[wid], idx_v)
            pltpu.async_copy(table_hbm.at[idx_v], rows_v, sem).wait()
            pltpu.sync_copy(rows_v, out_hbm.at[wid])

        _G_CACHE.append(_g)
    return _G_CACHE[0](table, flat_idx)


def _d_body(sub_ref, idx_ref, h_ref, hs_ref, inw_ref, inb_ref,
            outw_ref, outb_ref, ng_ref, nb_ref, wih_ref, whh_ref,
            bih_ref, bhh_ref, se_ref, cpw_ref,
            logits_ref, pred_ref, hn_ref, hsn_ref):
    hs = hs_ref[...]
    h = h_ref[...]
    subr = sub_ref[...].reshape(B * M, D)
    p = jax.lax.dot_general(_bf(subr), _bf(cpw_ref[...]),
                            (((1,), (1,)), ((), ())),
                            preferred_element_type=jnp.float32)
    n2 = jnp.sum(p * p, axis=1, keepdims=True)
    cs = p / jnp.maximum(jnp.sqrt(n2), 1e-12)
    csn = _l2k(cs)
    cs3 = cs.reshape(B, M, H)
    csn3 = csn.reshape(B, M, H)

    b2q = inb_ref[pl.ds(0, 1), 0:H]
    b2k = inb_ref[pl.ds(0, 1), H:2 * H]
    b2v = inb_ref[pl.ds(0, 1), 2 * H:3 * H]
    qp2 = _bdt(hs, inw_ref[0:H, :]) + b2q
    kp2 = jax.lax.dot_general(_bf(cs), _bf(inw_ref[H:2 * H, :]),
                              (((1,), (1,)), ((), ())),
                              preferred_element_type=jnp.float32) + b2k
    vp2 = jax.lax.dot_general(_bf(cs), _bf(inw_ref[2 * H:3 * H, :]),
                              (((1,), (1,)), ((), ())),
                              preferred_element_type=jnp.float32) + b2v
    s2 = _bmulsum(qp2[:, None, :], kp2.reshape(B, M, H), 2) * SCALE
    s2 = s2 - jnp.max(s2, axis=1, keepdims=True)
    e2 = jnp.exp(s2)
    att = e2 / jnp.sum(e2, axis=1, keepdims=True)
    o = _bmulsum(att[:, :, None], vp2.reshape(B, M, H), 1)
    a2 = _bdt(o, outw_ref[...]) + outb_ref[pl.ds(0, 1), :]
    x2 = a2 + hs
    rq = _l2k(_lnk(x2, ng_ref[pl.ds(0, 1), :], nb_ref[pl.ds(0, 1), :]))
    rt = _bmulsum(rq[:, None, :], csn3, 2) / TEMP

    idx = idx_ref[...]
    maxr = jnp.max(rt, axis=1, keepdims=True)
    eqr = rt == maxr
    pred = jnp.min(jnp.where(eqr, idx, jnp.int32(1 << 30)), axis=1,
                   keepdims=True)
    pred_ref[...] = jnp.broadcast_to(pred, (B, 128))
    onehot = (idx == pred).astype(jnp.float32)
    sel = jnp.sum(cs3 * onehot[:, :, None], axis=1)

    gi = _bdt(sel, wih_ref[...]) + bih_ref[pl.ds(0, 1), :]
    gh = _bdt(h, whh_ref[...]) + bhh_ref[pl.ds(0, 1), :]
    i_r, i_z, i_n = gi[:, 0:H], gi[:, H:2 * H], gi[:, 2 * H:3 * H]
    h_r, h_z, h_n = gh[:, 0:H], gh[:, H:2 * H], gh[:, 2 * H:3 * H]
    r = jax.nn.sigmoid(i_r + h_r)
    z = jax.nn.sigmoid(i_z + h_z)
    ngate = jnp.tanh(i_n + r * h_n)
    hn = _l2k((1.0 - z) * ngate + z * h)
    hsn = _l2k(hn + se_ref[pl.ds(1, 1), :])
    hn_ref[...] = hn
    hsn_ref[...] = hsn

    ki = jax.lax.broadcasted_iota(jnp.int32, (B, K), 1)
    acc = jnp.full((B, K), -1e4, jnp.float32)
    for m in range(M):
        acc = jnp.where(ki == idx[:, m:m + 1], rt[:, m:m + 1], acc)
    logits_ref[...] = acc


def _run_d(sub, idx, h, hs, inw, inb, outw, outb, ng, nb,
           wih, whh, bih, bhh, se, cpw):
    return pl.pallas_call(
        _d_body,
        out_shape=[
            jax.ShapeDtypeStruct((B, K), jnp.float32),
            jax.ShapeDtypeStruct((B, 128), jnp.int32),
            jax.ShapeDtypeStruct((B, H), jnp.float32),
            jax.ShapeDtypeStruct((B, H), jnp.float32),
        ],
    )(sub, idx, h, hs, inw, inb.reshape(1, 3 * H), outw,
      outb.reshape(1, H), ng.reshape(1, H), nb.reshape(1, H),
      wih, whh, bih.reshape(1, 3 * H), bhh.reshape(1, 3 * H), se, cpw)


def kernel(query_emb, cand_emb, attn_in_w, attn_in_b, attn_out_w, attn_out_b,
           norm_g, norm_b, query_proj_w, cand_proj_w, ref_in_w, ref_in_b,
           ref_out_w, ref_out_b, ref_norm_g, ref_norm_b, gru_w_ih, gru_w_hh,
           gru_b_ih, gru_b_hh, step_emb_w):
    cand_flat = cand_emb.reshape(B * K, D)

    cb, s = _run_a(cand_emb, query_emb, attn_in_w, attn_in_b)
    o = _run_a2(s, cb, attn_in_w, attn_in_b)
    h, hs = _run_b(o, query_emb, attn_out_w, attn_out_b, norm_g, norm_b,
                   query_proj_w, step_emb_w)

    logits_list = []
    prev_pred = jnp.full((B, 128), -1, jnp.int32)
    for _ in range(S):
        cheap = _run_c(cb, hs, cand_proj_w).reshape(B, K)
        idx, flat = _run_t(cheap, prev_pred)
        sub = _g_kernel(cand_flat, flat)
        logits, prev_pred, h, hs = _run_d(
            sub, idx, h, hs, ref_in_w, ref_in_b, ref_out_w, ref_out_b,
            ref_norm_g, ref_norm_b, gru_w_ih, gru_w_hh, gru_b_ih, gru_b_hh,
            step_emb_w, cand_proj_w)
        logits_list.append(logits)

    return jnp.stack(logits_list, axis=1)


# speedup vs baseline: 1.4395x; 1.4395x over previous
import functools

import jax
import jax.numpy as jnp
from jax import lax
from jax.experimental import pallas as pl
from jax.experimental.pallas import tpu as pltpu
from jax.experimental.pallas import tpu_sc as plsc

B, K, D, H, S, M = 32, 8192, 256, 256, 2, 64
TEMP = 0.1
KC = 2048
KB = K // KC
SCALE = 1.0 / 16.0

_HI = jax.lax.Precision.HIGHEST


def _bf(x):
    return x.astype(jnp.bfloat16)


def _dt(a, b):
    return jax.lax.dot_general(a, b, (((1,), (1,)), ((), ())), precision=_HI)


def _bdt(a, b):
    return jax.lax.dot_general(_bf(a), _bf(b), (((1,), (1,)), ((), ())),
                               preferred_element_type=jnp.float32)


def _bmulsum(a, b, axis):
    return jnp.sum(_bf(a).astype(jnp.float32) * _bf(b).astype(jnp.float32),
                   axis=axis)


def _l2k(x, axis=-1):
    n = jnp.sqrt(jnp.sum(x * x, axis=axis, keepdims=True))
    return x / jnp.maximum(n, 1e-12)


def _lnk(x, g, b):
    mu = jnp.mean(x, axis=-1, keepdims=True)
    xc = x - mu
    var = jnp.mean(xc * xc, axis=-1, keepdims=True)
    return xc / jnp.sqrt(var + 1e-5) * g + b


def _a_body(cand_ref, q_ref, inw_ref, inb_ref, cb_ref, s_ref, qp_scr):
    b = pl.program_id(0)
    kb = pl.program_id(1)

    @pl.when(kb == 0)
    def _init():
        qrow = q_ref[pl.ds(b, 1), :]
        qp_scr[...] = _dt(qrow, inw_ref[0:H, :]) + inb_ref[pl.ds(0, 1), 0:H]

    x = cand_ref[0]
    xb = _bf(x)
    cb_ref[0] = xb
    kp = jax.lax.dot_general(xb, _bf(inw_ref[H:2 * H, :]),
                             (((1,), (1,)), ((), ())),
                             preferred_element_type=jnp.float32) \
        + inb_ref[pl.ds(0, 1), H:2 * H]
    s = _bdt(qp_scr[...], kp) * SCALE
    s_ref[0, 0, :] = s[0]


def _run_a(cand, q, inw, inb):
    return pl.pallas_call(
        _a_body,
        grid=(B, KB),
        in_specs=[
            pl.BlockSpec((1, KC, D), lambda b, kb: (b, kb, 0)),
            pl.BlockSpec((B, D), lambda b, kb: (0, 0)),
            pl.BlockSpec((3 * H, H), lambda b, kb: (0, 0)),
            pl.BlockSpec((1, 3 * H), lambda b, kb: (0, 0)),
        ],
        out_specs=[
            pl.BlockSpec((1, KC, D), lambda b, kb: (b, kb, 0)),
            pl.BlockSpec((1, 1, KC), lambda b, kb: (b * KB + kb, 0, 0)),
        ],
        out_shape=[
            jax.ShapeDtypeStruct((B, K, D), jnp.bfloat16),
            jax.ShapeDtypeStruct((B * KB, 1, KC), jnp.float32),
        ],
        scratch_shapes=[pltpu.VMEM((1, H), jnp.float32)],
        compiler_params=pltpu.CompilerParams(
            dimension_semantics=("parallel", "arbitrary")),
    )(cand, q, inw, inb.reshape(1, 3 * H))


def _a2_body(srow_ref, sblk_ref, cb_ref, inw_ref, inb_ref, o_ref,
             ml_scr, acc_scr):
    kb = pl.program_id(1)

    @pl.when(kb == 0)
    def _init():
        srow = srow_ref[0, 0, :]
        m = jnp.max(srow)
        ml_scr[0] = m
        ml_scr[1] = jnp.sum(jnp.exp(srow - m))
        acc_scr[...] = jnp.zeros_like(acc_scr)

    att = jnp.exp(sblk_ref[0, 0, :] - ml_scr[0]) / ml_scr[1]
    vp = jax.lax.dot_general(cb_ref[0], _bf(inw_ref[2 * H:3 * H, :]),
                             (((1,), (1,)), ((), ())),
                             preferred_element_type=jnp.float32) \
        + inb_ref[pl.ds(0, 1), 2 * H:3 * H]
    acc_scr[...] += jax.lax.dot_general(
        _bf(att).reshape(1, KC), _bf(vp), (((1,), (0,)), ((), ())),
        preferred_element_type=jnp.float32)

    @pl.when(kb == KB - 1)
    def _fin():
        o_ref[0, 0, :] = acc_scr[0]


def _run_a2(s, cb, inw, inb):
    srow = s.reshape(B, 1, K)
    return pl.pallas_call(
        _a2_body,
        grid=(B, KB),
        in_specs=[
            pl.BlockSpec((1, 1, K), lambda b, kb: (b, 0, 0)),
            pl.BlockSpec((1, 1, KC), lambda b, kb: (b * KB + kb, 0, 0)),
            pl.BlockSpec((1, KC, D), lambda b, kb: (b, kb, 0)),
            pl.BlockSpec((3 * H, H), lambda b, kb: (0, 0)),
            pl.BlockSpec((1, 3 * H), lambda b, kb: (0, 0)),
        ],
        out_specs=pl.BlockSpec((1, 1, D), lambda b, kb: (b, 0, 0)),
        out_shape=jax.ShapeDtypeStruct((B, 1, D), jnp.float32),
        scratch_shapes=[
            pltpu.SMEM((2,), jnp.float32),
            pltpu.VMEM((1, H), jnp.float32),
        ],
        compiler_params=pltpu.CompilerParams(
            dimension_semantics=("parallel", "arbitrary")),
    )(srow, s, cb, inw, inb.reshape(1, 3 * H))


def _b_body(o_ref, q_ref, outw_ref, outb_ref, ng_ref, nb_ref,
            qpw_ref, se_ref, h_ref, hs_ref):
    o = o_ref[:, 0, :]
    a = _bdt(o, outw_ref[...]) + outb_ref[pl.ds(0, 1), :]
    x = a + q_ref[...]
    qe = _lnk(x, ng_ref[pl.ds(0, 1), :], nb_ref[pl.ds(0, 1), :])
    h = _l2k(_bdt(qe, qpw_ref[...]))
    hs = _l2k(h + se_ref[pl.ds(0, 1), :])
    h_ref[...] = h
    hs_ref[...] = hs


def _run_b(o, q, outw, outb, ng, nb, qpw, se):
    return pl.pallas_call(
        _b_body,
        out_shape=[jax.ShapeDtypeStruct((B, H), jnp.float32)] * 2,
    )(o, q, outw, outb.reshape(1, H), ng.reshape(1, H), nb.reshape(1, H),
      qpw, se)


def _c_body(cb_ref, hs_ref, cpw_ref, cheap_ref):
    p = jax.lax.dot_general(cb_ref[0], _bf(cpw_ref[...]),
                            (((1,), (1,)), ((), ())),
                            preferred_element_type=jnp.float32)
    n2 = jnp.sum(p * p, axis=1, keepdims=True)
    cpnb = _bf(p / jnp.maximum(jnp.sqrt(n2), 1e-12))
    hsb = _bf(hs_ref[0])
    dv = jax.lax.dot_general(hsb, cpnb, (((1,), (1,)), ((), ())),
                             preferred_element_type=jnp.float32)
    cheap_ref[0, 0, :] = dv[0] / TEMP


def _run_c(cb, hs, cpw):
    return pl.pallas_call(
        _c_body,
        grid=(B, KB),
        in_specs=[
            pl.BlockSpec((1, KC, D), lambda b, kb: (b, kb, 0)),
            pl.BlockSpec((1, 1, D), lambda b, kb: (b, 0, 0)),
            pl.BlockSpec((H, H), lambda b, kb: (0, 0)),
        ],
        out_specs=pl.BlockSpec((1, 1, KC), lambda b, kb: (b * KB + kb, 0, 0)),
        out_shape=jax.ShapeDtypeStruct((B * KB, 1, KC), jnp.float32),
        compiler_params=pltpu.CompilerParams(
            dimension_semantics=("parallel", "parallel")),
    )(cb, hs.reshape(B, 1, H), cpw)


def _t_body(cheap_ref, pp_ref, idx_ref, flat_ref):
    c = cheap_ref[...]
    ki = jax.lax.broadcasted_iota(jnp.int32, (B, K), 1)
    pp = pp_ref[:, 0:1]
    c = jnp.where(ki == pp, -1e9, c)
    cols = []
    for _ in range(M):
        v = jnp.max(c, axis=1, keepdims=True)
        eq = c == v
        im = jnp.min(jnp.where(eq, ki, K), axis=1, keepdims=True)
        cols.append(im)
        c = jnp.where(ki == im, -jnp.inf, c)
    idx = jnp.concatenate(cols, axis=1)
    idx_ref[...] = idx
    row = jax.lax.broadcasted_iota(jnp.int32, (B, M), 0)
    flat_ref[...] = idx + row * K


def _run_t(cheap, prev_pred):
    return pl.pallas_call(
        _t_body,
        out_shape=[
            jax.ShapeDtypeStruct((B, M), jnp.int32),
            jax.ShapeDtypeStruct((B, M), jnp.int32),
        ],
    )(cheap, prev_pred)


_G_CACHE = []


def _g_kernel(table, flat_idx):
    if not _G_CACHE:
        mesh = plsc.VectorSubcoreMesh(core_axis_name="c", subcore_axis_name="s")
        nc = mesh.num_cores

        @functools.partial(
            pl.kernel,
            mesh=mesh,
            out_type=jax.ShapeDtypeStruct((B, M, D), jnp.float32),
            scratch_types=[
                pltpu.VMEM((M,), jnp.int32),
                pltpu.VMEM((M, D), jnp.float32),
                pltpu.SemaphoreType.DMA,
            ],
        )
        def _g(table_hbm, idx_hbm, out_hbm, idx_v, rows_v, sem):
            wid = lax.axis_index("s") * nc + lax.axis_index("c")
            pltpu.sync_copy(idx_hbm.at[wid], idx_v)
            pltpu.async_copy(table_hbm.at[idx_v], rows_v, sem).wait()
            pltpu.sync_copy(rows_v, out_hbm.at[wid])

        _G_CACHE.append(_g)
    return _G_CACHE[0](table, flat_idx)


def _d_body(sub_ref, idx_ref, h_ref, hs_ref, inw_ref, inb_ref,
            outw_ref, outb_ref, ng_ref, nb_ref, wih_ref, whh_ref,
            bih_ref, bhh_ref, se_ref, cpw_ref,
            logits_ref, pred_ref, hn_ref, hsn_ref):
    hs = hs_ref[...]
    h = h_ref[...]
    subr = sub_ref[...].reshape(B * M, D)
    p = jax.lax.dot_general(_bf(subr), _bf(cpw_ref[...]),
                            (((1,), (1,)), ((), ())),
                            preferred_element_type=jnp.float32)
    n2 = jnp.sum(p * p, axis=1, keepdims=True)
    cs = p / jnp.maximum(jnp.sqrt(n2), 1e-12)
    csn = _l2k(cs)
    cs3 = cs.reshape(B, M, H)
    csn3 = csn.reshape(B, M, H)

    b2q = inb_ref[pl.ds(0, 1), 0:H]
    b2k = inb_ref[pl.ds(0, 1), H:2 * H]
    b2v = inb_ref[pl.ds(0, 1), 2 * H:3 * H]
    qp2 = _bdt(hs, inw_ref[0:H, :]) + b2q
    kp2 = jax.lax.dot_general(_bf(cs), _bf(inw_ref[H:2 * H, :]),
                              (((1,), (1,)), ((), ())),
                              preferred_element_type=jnp.float32) + b2k
    vp2 = jax.lax.dot_general(_bf(cs), _bf(inw_ref[2 * H:3 * H, :]),
                              (((1,), (1,)), ((), ())),
                              preferred_element_type=jnp.float32) + b2v
    s2 = _bmulsum(qp2[:, None, :], kp2.reshape(B, M, H), 2) * SCALE
    s2 = s2 - jnp.max(s2, axis=1, keepdims=True)
    e2 = jnp.exp(s2)
    att = e2 / jnp.sum(e2, axis=1, keepdims=True)
    o = _bmulsum(att[:, :, None], vp2.reshape(B, M, H), 1)
    a2 = _bdt(o, outw_ref[...]) + outb_ref[pl.ds(0, 1), :]
    x2 = a2 + hs
    rq = _l2k(_lnk(x2, ng_ref[pl.ds(0, 1), :], nb_ref[pl.ds(0, 1), :]))
    rt = _bmulsum(rq[:, None, :], csn3, 2) / TEMP

    idx = idx_ref[...]
    maxr = jnp.max(rt, axis=1, keepdims=True)
    eqr = rt == maxr
    pred = jnp.min(jnp.where(eqr, idx, jnp.int32(1 << 30)), axis=1,
                   keepdims=True)
    pred_ref[...] = jnp.broadcast_to(pred, (B, 128))
    onehot = (idx == pred).astype(jnp.float32)
    sel = jnp.sum(cs3 * onehot[:, :, None], axis=1)

    gi = _bdt(sel, wih_ref[...]) + bih_ref[pl.ds(0, 1), :]
    gh = _bdt(h, whh_ref[...]) + bhh_ref[pl.ds(0, 1), :]
    i_r, i_z, i_n = gi[:, 0:H], gi[:, H:2 * H], gi[:, 2 * H:3 * H]
    h_r, h_z, h_n = gh[:, 0:H], gh[:, H:2 * H], gh[:, 2 * H:3 * H]
    r = jax.nn.sigmoid(i_r + h_r)
    z = jax.nn.sigmoid(i_z + h_z)
    ngate = jnp.tanh(i_n + r * h_n)
    hn = _l2k((1.0 - z) * ngate + z * h)
    hsn = _l2k(hn + se_ref[pl.ds(1, 1), :])
    hn_ref[...] = hn
    hsn_ref[...] = hsn

    ki = jax.lax.broadcasted_iota(jnp.int32, (B, K), 1)
    acc = jnp.full((B, K), -1e4, jnp.float32)
    for m in range(M):
        acc = jnp.where(ki == idx[:, m:m + 1], rt[:, m:m + 1], acc)
    logits_ref[...] = acc


def _run_d(sub, idx, h, hs, inw, inb, outw, outb, ng, nb,
           wih, whh, bih, bhh, se, cpw):
    return pl.pallas_call(
        _d_body,
        out_shape=[
            jax.ShapeDtypeStruct((B, K), jnp.float32),
            jax.ShapeDtypeStruct((B, 128), jnp.int32),
            jax.ShapeDtypeStruct((B, H), jnp.float32),
            jax.ShapeDtypeStruct((B, H), jnp.float32),
        ],
    )(sub, idx, h, hs, inw, inb.reshape(1, 3 * H), outw,
      outb.reshape(1, H), ng.reshape(1, H), nb.reshape(1, H),
      wih, whh, bih.reshape(1, 3 * H), bhh.reshape(1, 3 * H), se, cpw)


def kernel(query_emb, cand_emb, attn_in_w, attn_in_b, attn_out_w, attn_out_b,
           norm_g, norm_b, query_proj_w, cand_proj_w, ref_in_w, ref_in_b,
           ref_out_w, ref_out_b, ref_norm_g, ref_norm_b, gru_w_ih, gru_w_hh,
           gru_b_ih, gru_b_hh, step_emb_w):
    cand_flat = cand_emb.reshape(B * K, D)

    cb, s = _run_a(cand_emb, query_emb, attn_in_w, attn_in_b)
    o = _run_a2(s, cb, attn_in_w, attn_in_b)
    h, hs = _run_b(o, query_emb, attn_out_w, attn_out_b, norm_g, norm_b,
                   query_proj_w, step_emb_w)

    logits_list = []
    prev_pred = jnp.full((B, 128), -1, jnp.int32)
    for _ in range(S):
        cheap = _run_c(cb, hs, cand_proj_w).reshape(B, K)
        idx, flat = _run_t(cheap, prev_pred)
        sub = _g_kernel(cand_flat, flat)
        logits, prev_pred, h, hs = _run_d(
            sub, idx, h, hs, ref_in_w, ref_in_b, ref_out_w, ref_out_b,
            ref_norm_g, ref_norm_b, gru_w_ih, gru_w_hh, gru_b_ih, gru_b_hh,
            step_emb_w, cand_proj_w)
        logits_list.append(logits)

    return jnp.stack(logits_list, axis=1)


# speedup vs baseline: 1.8461x; 1.2825x over previous
import functools

import jax
import jax.numpy as jnp
from jax import lax
from jax.experimental import pallas as pl
from jax.experimental.pallas import tpu as pltpu
from jax.experimental.pallas import tpu_sc as plsc

B, K, D, H, S, M = 32, 8192, 256, 256, 2, 64
TEMP = 0.1
KC = 4096
KB = K // KC
SCALE = 1.0 / 16.0

_HI = jax.lax.Precision.HIGHEST


def _bf(x):
    return x.astype(jnp.bfloat16)


def _dt(a, b):
    return jax.lax.dot_general(a, b, (((1,), (1,)), ((), ())), precision=_HI)


def _bdt(a, b):
    return jax.lax.dot_general(_bf(a), _bf(b), (((1,), (1,)), ((), ())),
                               preferred_element_type=jnp.float32)


def _bmulsum(a, b, axis):
    return jnp.sum(_bf(a).astype(jnp.float32) * _bf(b).astype(jnp.float32),
                   axis=axis)


def _l2k(x, axis=-1):
    n = jnp.sqrt(jnp.sum(x * x, axis=axis, keepdims=True))
    return x / jnp.maximum(n, 1e-12)


def _lnk(x, g, b):
    mu = jnp.mean(x, axis=-1, keepdims=True)
    xc = x - mu
    var = jnp.mean(xc * xc, axis=-1, keepdims=True)
    return xc / jnp.sqrt(var + 1e-5) * g + b


def _a_body(cand_ref, q_ref, inw_ref, inb_ref, cb_ref, s_ref, qp_scr):
    b = pl.program_id(0)
    kb = pl.program_id(1)

    @pl.when(kb == 0)
    def _init():
        qrow = q_ref[pl.ds(b, 1), :]
        qp_scr[...] = _dt(qrow, inw_ref[0:H, :]) + inb_ref[pl.ds(0, 1), 0:H]

    x = cand_ref[0]
    xb = _bf(x)
    cb_ref[0] = xb
    kp = jax.lax.dot_general(xb, _bf(inw_ref[H:2 * H, :]),
                             (((1,), (1,)), ((), ())),
                             preferred_element_type=jnp.float32) \
        + inb_ref[pl.ds(0, 1), H:2 * H]
    s = _bdt(qp_scr[...], kp) * SCALE
    s_ref[0, 0, :] = s[0]


def _run_a(cand, q, inw, inb):
    return pl.pallas_call(
        _a_body,
        grid=(B, KB),
        in_specs=[
            pl.BlockSpec((1, KC, D), lambda b, kb: (b, kb, 0)),
            pl.BlockSpec((B, D), lambda b, kb: (0, 0)),
            pl.BlockSpec((3 * H, H), lambda b, kb: (0, 0)),
            pl.BlockSpec((1, 3 * H), lambda b, kb: (0, 0)),
        ],
        out_specs=[
            pl.BlockSpec((1, KC, D), lambda b, kb: (b, kb, 0)),
            pl.BlockSpec((1, 1, KC), lambda b, kb: (b * KB + kb, 0, 0)),
        ],
        out_shape=[
            jax.ShapeDtypeStruct((B, K, D), jnp.bfloat16),
            jax.ShapeDtypeStruct((B * KB, 1, KC), jnp.float32),
        ],
        scratch_shapes=[pltpu.VMEM((1, H), jnp.float32)],
        compiler_params=pltpu.CompilerParams(
            dimension_semantics=("parallel", "arbitrary")),
    )(cand, q, inw, inb.reshape(1, 3 * H))


def _a2_body(srow_ref, sblk_ref, cb_ref, inw_ref, inb_ref, o_ref,
             ml_scr, acc_scr):
    kb = pl.program_id(1)

    @pl.when(kb == 0)
    def _init():
        srow = srow_ref[0, 0, :]
        m = jnp.max(srow)
        ml_scr[0] = m
        ml_scr[1] = jnp.sum(jnp.exp(srow - m))
        acc_scr[...] = jnp.zeros_like(acc_scr)

    att = jnp.exp(sblk_ref[0, 0, :] - ml_scr[0]) / ml_scr[1]
    vp = jax.lax.dot_general(cb_ref[0], _bf(inw_ref[2 * H:3 * H, :]),
                             (((1,), (1,)), ((), ())),
                             preferred_element_type=jnp.float32) \
        + inb_ref[pl.ds(0, 1), 2 * H:3 * H]
    acc_scr[...] += jax.lax.dot_general(
        _bf(att).reshape(1, KC), _bf(vp), (((1,), (0,)), ((), ())),
        preferred_element_type=jnp.float32)

    @pl.when(kb == KB - 1)
    def _fin():
        o_ref[0, 0, :] = acc_scr[0]


def _run_a2(s, cb, inw, inb):
    srow = s.reshape(B, 1, K)
    return pl.pallas_call(
        _a2_body,
        grid=(B, KB),
        in_specs=[
            pl.BlockSpec((1, 1, K), lambda b, kb: (b, 0, 0)),
            pl.BlockSpec((1, 1, KC), lambda b, kb: (b * KB + kb, 0, 0)),
            pl.BlockSpec((1, KC, D), lambda b, kb: (b, kb, 0)),
            pl.BlockSpec((3 * H, H), lambda b, kb: (0, 0)),
            pl.BlockSpec((1, 3 * H), lambda b, kb: (0, 0)),
        ],
        out_specs=pl.BlockSpec((1, 1, D), lambda b, kb: (b, 0, 0)),
        out_shape=jax.ShapeDtypeStruct((B, 1, D), jnp.float32),
        scratch_shapes=[
            pltpu.SMEM((2,), jnp.float32),
            pltpu.VMEM((1, H), jnp.float32),
        ],
        compiler_params=pltpu.CompilerParams(
            dimension_semantics=("parallel", "arbitrary")),
    )(srow, s, cb, inw, inb.reshape(1, 3 * H))


def _b_body(o_ref, q_ref, outw_ref, outb_ref, ng_ref, nb_ref,
            qpw_ref, se_ref, h_ref, hs_ref):
    o = o_ref[:, 0, :]
    a = _bdt(o, outw_ref[...]) + outb_ref[pl.ds(0, 1), :]
    x = a + q_ref[...]
    qe = _lnk(x, ng_ref[pl.ds(0, 1), :], nb_ref[pl.ds(0, 1), :])
    h = _l2k(_bdt(qe, qpw_ref[...]))
    hs = _l2k(h + se_ref[pl.ds(0, 1), :])
    h_ref[...] = h
    hs_ref[...] = hs


def _run_b(o, q, outw, outb, ng, nb, qpw, se):
    return pl.pallas_call(
        _b_body,
        out_shape=[jax.ShapeDtypeStruct((B, H), jnp.float32)] * 2,
    )(o, q, outw, outb.reshape(1, H), ng.reshape(1, H), nb.reshape(1, H),
      qpw, se)


def _c_body(cb_ref, hs_ref, cpw_ref, cheap_ref):
    p = jax.lax.dot_general(cb_ref[0], _bf(cpw_ref[...]),
                            (((1,), (1,)), ((), ())),
                            preferred_element_type=jnp.float32)
    n2 = jnp.sum(p * p, axis=1, keepdims=True)
    cpnb = _bf(p / jnp.maximum(jnp.sqrt(n2), 1e-12))
    hsb = _bf(hs_ref[0])
    dv = jax.lax.dot_general(hsb, cpnb, (((1,), (1,)), ((), ())),
                             preferred_element_type=jnp.float32)
    cheap_ref[0, 0, :] = dv[0] / TEMP


def _run_c(cb, hs, cpw):
    return pl.pallas_call(
        _c_body,
        grid=(B, KB),
        in_specs=[
            pl.BlockSpec((1, KC, D), lambda b, kb: (b, kb, 0)),
            pl.BlockSpec((1, 1, D), lambda b, kb: (b, 0, 0)),
            pl.BlockSpec((H, H), lambda b, kb: (0, 0)),
        ],
        out_specs=pl.BlockSpec((1, 1, KC), lambda b, kb: (b * KB + kb, 0, 0)),
        out_shape=jax.ShapeDtypeStruct((B * KB, 1, KC), jnp.float32),
        compiler_params=pltpu.CompilerParams(
            dimension_semantics=("parallel", "parallel")),
    )(cb, hs.reshape(B, 1, H), cpw)


def _t_body(cheap_ref, pp_ref, idx_ref, flat_ref):
    c = cheap_ref[...]
    ki = jax.lax.broadcasted_iota(jnp.int32, (B, K), 1)
    pp = pp_ref[:, 0:1]
    c = jnp.where(ki == pp, -1e9, c)
    cols = []
    for _ in range(M):
        v = jnp.max(c, axis=1, keepdims=True)
        eq = c == v
        im = jnp.min(jnp.where(eq, ki, K), axis=1, keepdims=True)
        cols.append(im)
        c = jnp.where(ki == im, -jnp.inf, c)
    idx = jnp.concatenate(cols, axis=1)
    idx_ref[...] = idx
    row = jax.lax.broadcasted_iota(jnp.int32, (B, M), 0)
    flat_ref[...] = idx + row * K


def _run_t(cheap, prev_pred):
    return pl.pallas_call(
        _t_body,
        out_shape=[
            jax.ShapeDtypeStruct((B, M), jnp.int32),
            jax.ShapeDtypeStruct((B, M), jnp.int32),
        ],
    )(cheap, prev_pred)


_G_CACHE = []


def _g_kernel(table, flat_idx):
    if not _G_CACHE:
        mesh = plsc.VectorSubcoreMesh(core_axis_name="c", subcore_axis_name="s")
        nc = mesh.num_cores

        @functools.partial(
            pl.kernel,
            mesh=mesh,
            out_type=jax.ShapeDtypeStruct((B, M, D), jnp.float32),
            scratch_types=[
                pltpu.VMEM((M,), jnp.int32),
                pltpu.VMEM((M, D), jnp.float32),
                pltpu.SemaphoreType.DMA,
            ],
        )
        def _g(table_hbm, idx_hbm, out_hbm, idx_v, rows_v, sem):
            wid = lax.axis_index("s") * nc + lax.axis_index("c")
            pltpu.sync_copy(idx_hbm.at[wid], idx_v)
            pltpu.async_copy(table_hbm.at[idx_v], rows_v, sem).wait()
            pltpu.sync_copy(rows_v, out_hbm.at[wid])

        _G_CACHE.append(_g)
    return _G_CACHE[0](table, flat_idx)


def _d_body(sub_ref, idx_ref, h_ref, hs_ref, inw_ref, inb_ref,
            outw_ref, outb_ref, ng_ref, nb_ref, wih_ref, whh_ref,
            bih_ref, bhh_ref, se_ref, cpw_ref,
            logits_ref, pred_ref, hn_ref, hsn_ref):
    hs = hs_ref[...]
    h = h_ref[...]
    subr = sub_ref[...].reshape(B * M, D)
    p = jax.lax.dot_general(_bf(subr), _bf(cpw_ref[...]),
                            (((1,), (1,)), ((), ())),
                            preferred_element_type=jnp.float32)
    n2 = jnp.sum(p * p, axis=1, keepdims=True)
    cs = p / jnp.maximum(jnp.sqrt(n2), 1e-12)
    csn = _l2k(cs)
    cs3 = cs.reshape(B, M, H)
    csn3 = csn.reshape(B, M, H)

    b2q = inb_ref[pl.ds(0, 1), 0:H]
    b2k = inb_ref[pl.ds(0, 1), H:2 * H]
    b2v = inb_ref[pl.ds(0, 1), 2 * H:3 * H]
    qp2 = _bdt(hs, inw_ref[0:H, :]) + b2q
    kp2 = jax.lax.dot_general(_bf(cs), _bf(inw_ref[H:2 * H, :]),
                              (((1,), (1,)), ((), ())),
                              preferred_element_type=jnp.float32) + b2k
    vp2 = jax.lax.dot_general(_bf(cs), _bf(inw_ref[2 * H:3 * H, :]),
                              (((1,), (1,)), ((), ())),
                              preferred_element_type=jnp.float32) + b2v
    s2 = _bmulsum(qp2[:, None, :], kp2.reshape(B, M, H), 2) * SCALE
    s2 = s2 - jnp.max(s2, axis=1, keepdims=True)
    e2 = jnp.exp(s2)
    att = e2 / jnp.sum(e2, axis=1, keepdims=True)
    o = _bmulsum(att[:, :, None], vp2.reshape(B, M, H), 1)
    a2 = _bdt(o, outw_ref[...]) + outb_ref[pl.ds(0, 1), :]
    x2 = a2 + hs
    rq = _l2k(_lnk(x2, ng_ref[pl.ds(0, 1), :], nb_ref[pl.ds(0, 1), :]))
    rt = _bmulsum(rq[:, None, :], csn3, 2) / TEMP

    idx = idx_ref[...]
    maxr = jnp.max(rt, axis=1, keepdims=True)
    eqr = rt == maxr
    pred = jnp.min(jnp.where(eqr, idx, jnp.int32(1 << 30)), axis=1,
                   keepdims=True)
    pred_ref[...] = jnp.broadcast_to(pred, (B, 128))
    onehot = (idx == pred).astype(jnp.float32)
    sel = jnp.sum(cs3 * onehot[:, :, None], axis=1)

    gi = _bdt(sel, wih_ref[...]) + bih_ref[pl.ds(0, 1), :]
    gh = _bdt(h, whh_ref[...]) + bhh_ref[pl.ds(0, 1), :]
    i_r, i_z, i_n = gi[:, 0:H], gi[:, H:2 * H], gi[:, 2 * H:3 * H]
    h_r, h_z, h_n = gh[:, 0:H], gh[:, H:2 * H], gh[:, 2 * H:3 * H]
    r = jax.nn.sigmoid(i_r + h_r)
    z = jax.nn.sigmoid(i_z + h_z)
    ngate = jnp.tanh(i_n + r * h_n)
    hn = _l2k((1.0 - z) * ngate + z * h)
    hsn = _l2k(hn + se_ref[pl.ds(1, 1), :])
    hn_ref[...] = hn
    hsn_ref[...] = hsn

    ki = jax.lax.broadcasted_iota(jnp.int32, (B, K), 1)
    acc = jnp.full((B, K), -1e4, jnp.float32)
    for m in range(M):
        acc = jnp.where(ki == idx[:, m:m + 1], rt[:, m:m + 1], acc)
    logits_ref[...] = acc


def _run_d(sub, idx, h, hs, inw, inb, outw, outb, ng, nb,
           wih, whh, bih, bhh, se, cpw):
    return pl.pallas_call(
        _d_body,
        out_shape=[
            jax.ShapeDtypeStruct((B, K), jnp.float32),
            jax.ShapeDtypeStruct((B, 128), jnp.int32),
            jax.ShapeDtypeStruct((B, H), jnp.float32),
            jax.ShapeDtypeStruct((B, H), jnp.float32),
        ],
    )(sub, idx, h, hs, inw, inb.reshape(1, 3 * H), outw,
      outb.reshape(1, H), ng.reshape(1, H), nb.reshape(1, H),
      wih, whh, bih.reshape(1, 3 * H), bhh.reshape(1, 3 * H), se, cpw)


def kernel(query_emb, cand_emb, attn_in_w, attn_in_b, attn_out_w, attn_out_b,
           norm_g, norm_b, query_proj_w, cand_proj_w, ref_in_w, ref_in_b,
           ref_out_w, ref_out_b, ref_norm_g, ref_norm_b, gru_w_ih, gru_w_hh,
           gru_b_ih, gru_b_hh, step_emb_w):
    cand_flat = cand_emb.reshape(B * K, D)

    cb, s = _run_a(cand_emb, query_emb, attn_in_w, attn_in_b)
    o = _run_a2(s, cb, attn_in_w, attn_in_b)
    h, hs = _run_b(o, query_emb, attn_out_w, attn_out_b, norm_g, norm_b,
                   query_proj_w, step_emb_w)

    logits_list = []
    prev_pred = jnp.full((B, 128), -1, jnp.int32)
    for _ in range(S):
        cheap = _run_c(cb, hs, cand_proj_w).reshape(B, K)
        idx, flat = _run_t(cheap, prev_pred)
        sub = _g_kernel(cand_flat, flat)
        logits, prev_pred, h, hs = _run_d(
            sub, idx, h, hs, ref_in_w, ref_in_b, ref_out_w, ref_out_b,
            ref_norm_g, ref_norm_b, gru_w_ih, gru_w_hh, gru_b_ih, gru_b_hh,
            step_emb_w, cand_proj_w)
        logits_list.append(logits)

    return jnp.stack(logits_list, axis=1)


# speedup vs baseline: 2.1319x; 1.1548x over previous
import functools

import jax
import jax.numpy as jnp
from jax import lax
from jax.experimental import pallas as pl
from jax.experimental.pallas import tpu as pltpu
from jax.experimental.pallas import tpu_sc as plsc

B, K, D, H, S, M = 32, 8192, 256, 256, 2, 64
TEMP = 0.1
KC = 8192
KB = K // KC
SCALE = 1.0 / 16.0

_HI = jax.lax.Precision.HIGHEST


def _bf(x):
    return x.astype(jnp.bfloat16)


def _dt(a, b):
    return jax.lax.dot_general(a, b, (((1,), (1,)), ((), ())), precision=_HI)


def _bdt(a, b):
    return jax.lax.dot_general(_bf(a), _bf(b), (((1,), (1,)), ((), ())),
                               preferred_element_type=jnp.float32)


def _bmulsum(a, b, axis):
    return jnp.sum(_bf(a).astype(jnp.float32) * _bf(b).astype(jnp.float32),
                   axis=axis)


def _l2k(x, axis=-1):
    n = jnp.sqrt(jnp.sum(x * x, axis=axis, keepdims=True))
    return x / jnp.maximum(n, 1e-12)


def _lnk(x, g, b):
    mu = jnp.mean(x, axis=-1, keepdims=True)
    xc = x - mu
    var = jnp.mean(xc * xc, axis=-1, keepdims=True)
    return xc / jnp.sqrt(var + 1e-5) * g + b


def _a_body(cand_ref, q_ref, inw_ref, inb_ref, cb_ref, s_ref, qp_scr):
    b = pl.program_id(0)
    kb = pl.program_id(1)

    @pl.when(kb == 0)
    def _init():
        qrow = q_ref[pl.ds(b, 1), :]
        qp_scr[...] = _dt(qrow, inw_ref[0:H, :]) + inb_ref[pl.ds(0, 1), 0:H]

    x = cand_ref[0]
    xb = _bf(x)
    cb_ref[0] = xb
    kp = jax.lax.dot_general(xb, _bf(inw_ref[H:2 * H, :]),
                             (((1,), (1,)), ((), ())),
                             preferred_element_type=jnp.float32) \
        + inb_ref[pl.ds(0, 1), H:2 * H]
    s = _bdt(qp_scr[...], kp) * SCALE
    s_ref[0, 0, :] = s[0]


def _run_a(cand, q, inw, inb):
    return pl.pallas_call(
        _a_body,
        grid=(B, KB),
        in_specs=[
            pl.BlockSpec((1, KC, D), lambda b, kb: (b, kb, 0)),
            pl.BlockSpec((B, D), lambda b, kb: (0, 0)),
            pl.BlockSpec((3 * H, H), lambda b, kb: (0, 0)),
            pl.BlockSpec((1, 3 * H), lambda b, kb: (0, 0)),
        ],
        out_specs=[
            pl.BlockSpec((1, KC, D), lambda b, kb: (b, kb, 0)),
            pl.BlockSpec((1, 1, KC), lambda b, kb: (b * KB + kb, 0, 0)),
        ],
        out_shape=[
            jax.ShapeDtypeStruct((B, K, D), jnp.bfloat16),
            jax.ShapeDtypeStruct((B * KB, 1, KC), jnp.float32),
        ],
        scratch_shapes=[pltpu.VMEM((1, H), jnp.float32)],
        compiler_params=pltpu.CompilerParams(
            dimension_semantics=("parallel", "arbitrary")),
    )(cand, q, inw, inb.reshape(1, 3 * H))


def _a2_body(srow_ref, sblk_ref, cb_ref, inw_ref, inb_ref, o_ref,
             ml_scr, acc_scr):
    kb = pl.program_id(1)

    @pl.when(kb == 0)
    def _init():
        srow = srow_ref[0, 0, :]
        m = jnp.max(srow)
        ml_scr[0] = m
        ml_scr[1] = jnp.sum(jnp.exp(srow - m))
        acc_scr[...] = jnp.zeros_like(acc_scr)

    att = jnp.exp(sblk_ref[0, 0, :] - ml_scr[0]) / ml_scr[1]
    vp = jax.lax.dot_general(cb_ref[0], _bf(inw_ref[2 * H:3 * H, :]),
                             (((1,), (1,)), ((), ())),
                             preferred_element_type=jnp.float32) \
        + inb_ref[pl.ds(0, 1), 2 * H:3 * H]
    acc_scr[...] += jax.lax.dot_general(
        _bf(att).reshape(1, KC), _bf(vp), (((1,), (0,)), ((), ())),
        preferred_element_type=jnp.float32)

    @pl.when(kb == KB - 1)
    def _fin():
        o_ref[0, 0, :] = acc_scr[0]


def _run_a2(s, cb, inw, inb):
    srow = s.reshape(B, 1, K)
    return pl.pallas_call(
        _a2_body,
        grid=(B, KB),
        in_specs=[
            pl.BlockSpec((1, 1, K), lambda b, kb: (b, 0, 0)),
            pl.BlockSpec((1, 1, KC), lambda b, kb: (b * KB + kb, 0, 0)),
            pl.BlockSpec((1, KC, D), lambda b, kb: (b, kb, 0)),
            pl.BlockSpec((3 * H, H), lambda b, kb: (0, 0)),
            pl.BlockSpec((1, 3 * H), lambda b, kb: (0, 0)),
        ],
        out_specs=pl.BlockSpec((1, 1, D), lambda b, kb: (b, 0, 0)),
        out_shape=jax.ShapeDtypeStruct((B, 1, D), jnp.float32),
        scratch_shapes=[
            pltpu.SMEM((2,), jnp.float32),
            pltpu.VMEM((1, H), jnp.float32),
        ],
        compiler_params=pltpu.CompilerParams(
            dimension_semantics=("parallel", "arbitrary")),
    )(srow, s, cb, inw, inb.reshape(1, 3 * H))


def _b_body(o_ref, q_ref, outw_ref, outb_ref, ng_ref, nb_ref,
            qpw_ref, se_ref, h_ref, hs_ref):
    o = o_ref[:, 0, :]
    a = _bdt(o, outw_ref[...]) + outb_ref[pl.ds(0, 1), :]
    x = a + q_ref[...]
    qe = _lnk(x, ng_ref[pl.ds(0, 1), :], nb_ref[pl.ds(0, 1), :])
    h = _l2k(_bdt(qe, qpw_ref[...]))
    hs = _l2k(h + se_ref[pl.ds(0, 1), :])
    h_ref[...] = h
    hs_ref[...] = hs


def _run_b(o, q, outw, outb, ng, nb, qpw, se):
    return pl.pallas_call(
        _b_body,
        out_shape=[jax.ShapeDtypeStruct((B, H), jnp.float32)] * 2,
    )(o, q, outw, outb.reshape(1, H), ng.reshape(1, H), nb.reshape(1, H),
      qpw, se)


def _c_body(cb_ref, hs_ref, cpw_ref, cheap_ref):
    p = jax.lax.dot_general(cb_ref[0], _bf(cpw_ref[...]),
                            (((1,), (1,)), ((), ())),
                            preferred_element_type=jnp.float32)
    n2 = jnp.sum(p * p, axis=1, keepdims=True)
    cpnb = _bf(p / jnp.maximum(jnp.sqrt(n2), 1e-12))
    hsb = _bf(hs_ref[0])
    dv = jax.lax.dot_general(hsb, cpnb, (((1,), (1,)), ((), ())),
                             preferred_element_type=jnp.float32)
    cheap_ref[0, 0, :] = dv[0] / TEMP


def _run_c(cb, hs, cpw):
    return pl.pallas_call(
        _c_body,
        grid=(B, KB),
        in_specs=[
            pl.BlockSpec((1, KC, D), lambda b, kb: (b, kb, 0)),
            pl.BlockSpec((1, 1, D), lambda b, kb: (b, 0, 0)),
            pl.BlockSpec((H, H), lambda b, kb: (0, 0)),
        ],
        out_specs=pl.BlockSpec((1, 1, KC), lambda b, kb: (b * KB + kb, 0, 0)),
        out_shape=jax.ShapeDtypeStruct((B * KB, 1, KC), jnp.float32),
        compiler_params=pltpu.CompilerParams(
            dimension_semantics=("parallel", "parallel")),
    )(cb, hs.reshape(B, 1, H), cpw)


def _t_body(cheap_ref, pp_ref, idx_ref, flat_ref):
    c = cheap_ref[...]
    ki = jax.lax.broadcasted_iota(jnp.int32, (B, K), 1)
    pp = pp_ref[:, 0:1]
    c = jnp.where(ki == pp, -1e9, c)
    cols = []
    for _ in range(M):
        v = jnp.max(c, axis=1, keepdims=True)
        eq = c == v
        im = jnp.min(jnp.where(eq, ki, K), axis=1, keepdims=True)
        cols.append(im)
        c = jnp.where(ki == im, -jnp.inf, c)
    idx = jnp.concatenate(cols, axis=1)
    idx_ref[...] = idx
    row = jax.lax.broadcasted_iota(jnp.int32, (B, M), 0)
    flat_ref[...] = idx + row * K


def _run_t(cheap, prev_pred):
    return pl.pallas_call(
        _t_body,
        out_shape=[
            jax.ShapeDtypeStruct((B, M), jnp.int32),
            jax.ShapeDtypeStruct((B, M), jnp.int32),
        ],
    )(cheap, prev_pred)


_G_CACHE = []


def _g_kernel(table, flat_idx):
    if not _G_CACHE:
        mesh = plsc.VectorSubcoreMesh(core_axis_name="c", subcore_axis_name="s")
        nc = mesh.num_cores

        @functools.partial(
            pl.kernel,
            mesh=mesh,
            out_type=jax.ShapeDtypeStruct((B, M, D), jnp.float32),
            scratch_types=[
                pltpu.VMEM((M,), jnp.int32),
                pltpu.VMEM((M, D), jnp.float32),
                pltpu.SemaphoreType.DMA,
            ],
        )
        def _g(table_hbm, idx_hbm, out_hbm, idx_v, rows_v, sem):
            wid = lax.axis_index("s") * nc + lax.axis_index("c")
            pltpu.sync_copy(idx_hbm.at[wid], idx_v)
            pltpu.async_copy(table_hbm.at[idx_v], rows_v, sem).wait()
            pltpu.sync_copy(rows_v, out_hbm.at[wid])

        _G_CACHE.append(_g)
    return _G_CACHE[0](table, flat_idx)


def _d_body(sub_ref, idx_ref, h_ref, hs_ref, inw_ref, inb_ref,
            outw_ref, outb_ref, ng_ref, nb_ref, wih_ref, whh_ref,
            bih_ref, bhh_ref, se_ref, cpw_ref,
            logits_ref, pred_ref, hn_ref, hsn_ref):
    hs = hs_ref[...]
    h = h_ref[...]
    subr = sub_ref[...].reshape(B * M, D)
    p = jax.lax.dot_general(_bf(subr), _bf(cpw_ref[...]),
                            (((1,), (1,)), ((), ())),
                            preferred_element_type=jnp.float32)
    n2 = jnp.sum(p * p, axis=1, keepdims=True)
    cs = p / jnp.maximum(jnp.sqrt(n2), 1e-12)
    csn = _l2k(cs)
    cs3 = cs.reshape(B, M, H)
    csn3 = csn.reshape(B, M, H)

    b2q = inb_ref[pl.ds(0, 1), 0:H]
    b2k = inb_ref[pl.ds(0, 1), H:2 * H]
    b2v = inb_ref[pl.ds(0, 1), 2 * H:3 * H]
    qp2 = _bdt(hs, inw_ref[0:H, :]) + b2q
    kp2 = jax.lax.dot_general(_bf(cs), _bf(inw_ref[H:2 * H, :]),
                              (((1,), (1,)), ((), ())),
                              preferred_element_type=jnp.float32) + b2k
    vp2 = jax.lax.dot_general(_bf(cs), _bf(inw_ref[2 * H:3 * H, :]),
                              (((1,), (1,)), ((), ())),
                              preferred_element_type=jnp.float32) + b2v
    s2 = _bmulsum(qp2[:, None, :], kp2.reshape(B, M, H), 2) * SCALE
    s2 = s2 - jnp.max(s2, axis=1, keepdims=True)
    e2 = jnp.exp(s2)
    att = e2 / jnp.sum(e2, axis=1, keepdims=True)
    o = _bmulsum(att[:, :, None], vp2.reshape(B, M, H), 1)
    a2 = _bdt(o, outw_ref[...]) + outb_ref[pl.ds(0, 1), :]
    x2 = a2 + hs
    rq = _l2k(_lnk(x2, ng_ref[pl.ds(0, 1), :], nb_ref[pl.ds(0, 1), :]))
    rt = _bmulsum(rq[:, None, :], csn3, 2) / TEMP

    idx = idx_ref[...]
    maxr = jnp.max(rt, axis=1, keepdims=True)
    eqr = rt == maxr
    pred = jnp.min(jnp.where(eqr, idx, jnp.int32(1 << 30)), axis=1,
                   keepdims=True)
    pred_ref[...] = jnp.broadcast_to(pred, (B, 128))
    onehot = (idx == pred).astype(jnp.float32)
    sel = jnp.sum(cs3 * onehot[:, :, None], axis=1)

    gi = _bdt(sel, wih_ref[...]) + bih_ref[pl.ds(0, 1), :]
    gh = _bdt(h, whh_ref[...]) + bhh_ref[pl.ds(0, 1), :]
    i_r, i_z, i_n = gi[:, 0:H], gi[:, H:2 * H], gi[:, 2 * H:3 * H]
    h_r, h_z, h_n = gh[:, 0:H], gh[:, H:2 * H], gh[:, 2 * H:3 * H]
    r = jax.nn.sigmoid(i_r + h_r)
    z = jax.nn.sigmoid(i_z + h_z)
    ngate = jnp.tanh(i_n + r * h_n)
    hn = _l2k((1.0 - z) * ngate + z * h)
    hsn = _l2k(hn + se_ref[pl.ds(1, 1), :])
    hn_ref[...] = hn
    hsn_ref[...] = hsn

    ki = jax.lax.broadcasted_iota(jnp.int32, (B, K), 1)
    acc = jnp.full((B, K), -1e4, jnp.float32)
    for m in range(M):
        acc = jnp.where(ki == idx[:, m:m + 1], rt[:, m:m + 1], acc)
    logits_ref[...] = acc


def _run_d(sub, idx, h, hs, inw, inb, outw, outb, ng, nb,
           wih, whh, bih, bhh, se, cpw):
    return pl.pallas_call(
        _d_body,
        out_shape=[
            jax.ShapeDtypeStruct((B, K), jnp.float32),
            jax.ShapeDtypeStruct((B, 128), jnp.int32),
            jax.ShapeDtypeStruct((B, H), jnp.float32),
            jax.ShapeDtypeStruct((B, H), jnp.float32),
        ],
    )(sub, idx, h, hs, inw, inb.reshape(1, 3 * H), outw,
      outb.reshape(1, H), ng.reshape(1, H), nb.reshape(1, H),
      wih, whh, bih.reshape(1, 3 * H), bhh.reshape(1, 3 * H), se, cpw)


def kernel(query_emb, cand_emb, attn_in_w, attn_in_b, attn_out_w, attn_out_b,
           norm_g, norm_b, query_proj_w, cand_proj_w, ref_in_w, ref_in_b,
           ref_out_w, ref_out_b, ref_norm_g, ref_norm_b, gru_w_ih, gru_w_hh,
           gru_b_ih, gru_b_hh, step_emb_w):
    cand_flat = cand_emb.reshape(B * K, D)

    cb, s = _run_a(cand_emb, query_emb, attn_in_w, attn_in_b)
    o = _run_a2(s, cb, attn_in_w, attn_in_b)
    h, hs = _run_b(o, query_emb, attn_out_w, attn_out_b, norm_g, norm_b,
                   query_proj_w, step_emb_w)

    logits_list = []
    prev_pred = jnp.full((B, 128), -1, jnp.int32)
    for _ in range(S):
        cheap = _run_c(cb, hs, cand_proj_w).reshape(B, K)
        idx, flat = _run_t(cheap, prev_pred)
        sub = _g_kernel(cand_flat, flat)
        logits, prev_pred, h, hs = _run_d(
            sub, idx, h, hs, ref_in_w, ref_in_b, ref_out_w, ref_out_b,
            ref_norm_g, ref_norm_b, gru_w_ih, gru_w_hh, gru_b_ih, gru_b_hh,
            step_emb_w, cand_proj_w)
        logits_list.append(logits)

    return jnp.stack(logits_list, axis=1)
